# Initial kernel scaffold; baseline (speedup 1.0000x reference)
import jax
import jax.numpy as jnp
from jax.experimental import pallas as pl


def kernel(x, edge_index, enc_W, enc_b, ln1_g, ln1_b, t, msg_scale, W1, b1, ln2_g, ln2_b, W2, b2, fn_g, fn_b):
    raise NotImplementedError("write your pallas kernel here")


# speedup vs baseline: 7.1641x; 7.1641x over previous
import functools

import jax
import jax.numpy as jnp
from jax import lax
from jax.experimental import pallas as pl
from jax.experimental.pallas import tpu as pltpu
from jax.experimental.pallas import tpu_sc as plsc

N = 10000
E = 320000
D = 128
L = 14
GEN_EPS = 1e-7

NC = 2
NS = 16
EP = E // NS
EB = 80
NBATCH = EP // EB
RPT = 624
RTAIL = N - NS * RPT

BN = 1000


def _sc_agg_body(table_hbm, srcx_hbm, dst_hbm, out_hbm,
                 gidx, sidx, rows_v, zbuf, acc_sh, sem):
    c = lax.axis_index("c")
    s = lax.axis_index("s")

    def _zb(r, carry):
        for k in range(D // 16):
            zbuf[r, pl.ds(k * 16, 16)] = jnp.zeros((16,), jnp.float32)
        return carry
    lax.fori_loop(0, 128, _zb, 0)
    for k in range(4):
        pltpu.sync_copy(zbuf, acc_sh.at[pl.ds(s * RPT + k * 128, 128)])
    pltpu.sync_copy(zbuf.at[pl.ds(0, RPT - 512)],
                    acc_sh.at[pl.ds(s * RPT + 512, RPT - 512)])

    @pl.when(s == 0)
    def _():
        pltpu.sync_copy(zbuf.at[pl.ds(0, RTAIL)],
                        acc_sh.at[pl.ds(NS * RPT, RTAIL)])

    plsc.subcore_barrier()

    def _step(j, carry):
        base = s * EP + j * EB
        pltpu.sync_copy(srcx_hbm.at[pl.ds(c * E + base, EB)], gidx)
        pltpu.sync_copy(dst_hbm.at[pl.ds(base, EB)], sidx)
        pltpu.async_copy(table_hbm.at[gidx], rows_v, sem).wait()
        pltpu.sync_copy(rows_v, acc_sh.at[sidx], add=True)
        return carry
    lax.fori_loop(0, NBATCH, _step, 0)

    plsc.subcore_barrier()
    pltpu.sync_copy(acc_sh.at[pl.ds(s * RPT, RPT)],
                    out_hbm.at[c, pl.ds(s * RPT, RPT)])

    @pl.when(s == 0)
    def _():
        pltpu.sync_copy(acc_sh.at[pl.ds(NS * RPT, RTAIL)],
                        out_hbm.at[c, pl.ds(NS * RPT, RTAIL)])


@functools.lru_cache(maxsize=None)
def _make_sc_agg():
    mesh = plsc.VectorSubcoreMesh(core_axis_name="c", subcore_axis_name="s")
    return pl.kernel(
        _sc_agg_body,
        out_type=jax.ShapeDtypeStruct((NC, N, D), jnp.float32),
        mesh=mesh,
        scratch_types=[
            pltpu.VMEM((EB,), jnp.int32),
            pltpu.VMEM((EB,), jnp.int32),
            pltpu.VMEM((EB, D), jnp.float32),
            pltpu.VMEM((128, D), jnp.float32),
            pltpu.VMEM_SHARED((N, D), jnp.float32),
            pltpu.SemaphoreType.DMA,
        ],
    )


def _aggregate(table2, srcx, dst):
    return _make_sc_agg()(table2, srcx, dst)


def _ln(h, g, b, eps=1e-5):
    mu = jnp.mean(h, axis=-1, keepdims=True)
    d = h - mu
    v = jnp.mean(d * d, axis=-1, keepdims=True)
    return d / jnp.sqrt(v + eps) * g + b


def _enc_body(x_ref, w_ref, b_ref, o_ref):
    o_ref[...] = jnp.dot(x_ref[...], w_ref[...],
                         preferred_element_type=jnp.float32) + b_ref[...]


@functools.lru_cache(maxsize=None)
def _make_enc():
    return pl.pallas_call(
        _enc_body,
        grid=(N // BN,),
        in_specs=[
            pl.BlockSpec((BN, D), lambda i: (i, 0)),
            pl.BlockSpec((D, D), lambda i: (0, 0)),
            pl.BlockSpec((1, D), lambda i: (0, 0)),
        ],
        out_specs=pl.BlockSpec((BN, D), lambda i: (i, 0)),
        out_shape=jax.ShapeDtypeStruct((N, D), jnp.float32),
    )


def _sa_body(h_ref, g_ref, b_ref, t_ref, tab_ref, hn_ref):
    hn = _ln(h_ref[...], g_ref[...], b_ref[...])
    hn = jnp.where(hn >= 0, hn, 0.01 * hn)
    hn_ref[...] = hn
    m = jnp.maximum(hn, 0.0) + GEN_EPS
    em = jnp.exp(m * t_ref[...])
    p = m * em
    tab_ref[0] = jnp.concatenate([em[:, :64], p[:, :64]], axis=1)
    tab_ref[1] = jnp.concatenate([em[:, 64:], p[:, 64:]], axis=1)


@functools.lru_cache(maxsize=None)
def _make_stage_a():
    return pl.pallas_call(
        _sa_body,
        grid=(N // BN,),
        in_specs=[
            pl.BlockSpec((BN, D), lambda i: (i, 0)),
            pl.BlockSpec((1, D), lambda i: (0, 0)),
            pl.BlockSpec((1, D), lambda i: (0, 0)),
            pl.BlockSpec((1, D), lambda i: (0, 0)),
        ],
        out_specs=[
            pl.BlockSpec((NC, BN, D), lambda i: (0, i, 0)),
            pl.BlockSpec((BN, D), lambda i: (i, 0)),
        ],
        out_shape=[
            jax.ShapeDtypeStruct((NC, N, D), jnp.float32),
            jax.ShapeDtypeStruct((N, D), jnp.float32),
        ],
    )


def _sb_body(acc_ref, h_ref, hn_ref, w1_ref, b1_ref, g2_ref, bb2_ref,
             w2_ref, b2_ref, ms_ref, o_ref):
    a0 = acc_ref[0]
    a1 = acc_ref[1]
    den = jnp.concatenate([a0[:, :64], a1[:, :64]], axis=1)
    num = jnp.concatenate([a0[:, 64:], a1[:, 64:]], axis=1)
    out = num / (den + 1e-16)
    nrm = jnp.sqrt(jnp.sum(out * out, axis=-1, keepdims=True))
    msg_n = out / jnp.maximum(nrm, 1e-12)
    hn = hn_ref[...]
    x_norm = jnp.sqrt(jnp.sum(hn * hn, axis=-1, keepdims=True))
    out = msg_n * x_norm * ms_ref[...] + hn
    z = jnp.dot(out, w1_ref[...], preferred_element_type=jnp.float32) + b1_ref[...]
    z = _ln(z, g2_ref[...], bb2_ref[...])
    z = jnp.maximum(z, 0.0)
    y = jnp.dot(z, w2_ref[...], preferred_element_type=jnp.float32) + b2_ref[...]
    o_ref[...] = h_ref[...] + y


@functools.lru_cache(maxsize=None)
def _make_stage_b():
    return pl.pallas_call(
        _sb_body,
        grid=(N // BN,),
        in_specs=[
            pl.BlockSpec((NC, BN, D), lambda i: (0, i, 0)),
            pl.BlockSpec((BN, D), lambda i: (i, 0)),
            pl.BlockSpec((BN, D), lambda i: (i, 0)),
            pl.BlockSpec((D, 2 * D), lambda i: (0, 0)),
            pl.BlockSpec((1, 2 * D), lambda i: (0, 0)),
            pl.BlockSpec((1, 2 * D), lambda i: (0, 0)),
            pl.BlockSpec((1, 2 * D), lambda i: (0, 0)),
            pl.BlockSpec((2 * D, D), lambda i: (0, 0)),
            pl.BlockSpec((1, D), lambda i: (0, 0)),
            pl.BlockSpec((1, D), lambda i: (0, 0)),
        ],
        out_specs=pl.BlockSpec((BN, D), lambda i: (i, 0)),
        out_shape=jax.ShapeDtypeStruct((N, D), jnp.float32),
    )


def _fin_body(h_ref, g_ref, b_ref, o_ref):
    i = pl.program_id(0)
    hh = _ln(h_ref[...], g_ref[...], b_ref[...])
    hh = jnp.where(hh >= 0, hh, 0.01 * hh)
    part = jnp.sum(hh, axis=0, keepdims=True) * (1.0 / N)

    @pl.when(i == 0)
    def _():
        o_ref[...] = part

    @pl.when(i != 0)
    def _():
        o_ref[...] = o_ref[...] + part


@functools.lru_cache(maxsize=None)
def _make_final():
    return pl.pallas_call(
        _fin_body,
        grid=(N // BN,),
        in_specs=[
            pl.BlockSpec((BN, D), lambda i: (i, 0)),
            pl.BlockSpec((1, D), lambda i: (0, 0)),
            pl.BlockSpec((1, D), lambda i: (0, 0)),
        ],
        out_specs=pl.BlockSpec((1, D), lambda i: (0, 0)),
        out_shape=jax.ShapeDtypeStruct((1, D), jnp.float32),
    )


def kernel(x, edge_index, enc_W, enc_b, ln1_g, ln1_b, t, msg_scale,
           W1, b1, ln2_g, ln2_b, W2, b2, fn_g, fn_b):
    src = edge_index[0]
    dst = edge_index[1]
    srcx = jnp.concatenate([src, src + N])

    h = _make_enc()(x, enc_W, enc_b.reshape(1, D))
    stage_a = _make_stage_a()
    stage_b = _make_stage_b()
    for i in range(L):
        t_b = jnp.full((1, D), t[i], jnp.float32)
        ms_b = jnp.full((1, D), msg_scale[i], jnp.float32)
        tab, hn = stage_a(h, ln1_g[i].reshape(1, D), ln1_b[i].reshape(1, D), t_b)
        acc = _aggregate(tab.reshape(2 * N, D), srcx, dst)
        h = stage_b(acc, h, hn,
                    W1[i], b1[i].reshape(1, 2 * D),
                    ln2_g[i].reshape(1, 2 * D), ln2_b[i].reshape(1, 2 * D),
                    W2[i], b2[i].reshape(1, D), ms_b)
    return _make_final()(h, fn_g.reshape(1, D), fn_b.reshape(1, D))
